# Initial kernel scaffold; baseline (speedup 1.0000x reference)
import jax
import jax.numpy as jnp
from jax.experimental import pallas as pl


def kernel(node_embedding, edge_index, edge_type, W_rel0, W_root0, b0, g0, be0, W_rel1, W_root1, b1, g1, be1):
    raise NotImplementedError("write your pallas kernel here")


# speedup vs baseline: 4.0844x; 4.0844x over previous
import jax
import jax.numpy as jnp
from jax import lax
from jax.experimental import pallas as pl
from jax.experimental.pallas import tpu as pltpu
from jax.experimental.pallas import tpu_sc as plsc

_NC = 2
_NS = 16
_NT = _NC * _NS
_LANE = 16
_BLK = 128
_CW = 32


def _make_sc_agg(n_nodes, n_rel, d, epb, with_counts):
    nch = d // _CW
    rps = -(-(n_rel * n_nodes + 1) // (_NS * _BLK)) * _BLK
    acc_rows = rps * _NS

    mesh = plsc.VectorSubcoreMesh(core_axis_name="c", subcore_axis_name="s")
    out_type = [jax.ShapeDtypeStruct((nch, _NC, acc_rows, _CW), jnp.float32)]
    if with_counts:
        out_type.append(jax.ShapeDtypeStruct((_NC, acc_rows, _CW), jnp.float32))

    scratch = [
        pltpu.VMEM((epb, _BLK), jnp.int32),
        pltpu.VMEM((8, _BLK), jnp.int32),
        pltpu.VMEM((8, _BLK), jnp.int32),
        pltpu.VMEM((epb, _BLK), jnp.int32),
        pltpu.VMEM((_BLK, _CW), jnp.float32),
        pltpu.VMEM_SHARED((acc_rows, _CW), jnp.float32),
    ]
    if with_counts:
        scratch += [
            pltpu.VMEM((_BLK, _CW), jnp.float32),
        ]

    def body(src_h, dst_h, typ_h, *rest):
        xs_h = rest[:nch]
        if with_counts:
            sums_h, cnt_h = rest[nch], rest[nch + 1]
            (src_v, dst_v, typ_v, sidx_v, rows_v, acc,
             ones_v) = rest[nch + 2:]
        else:
            sums_h = rest[nch]
            (src_v, dst_v, typ_v, sidx_v, rows_v, acc) = rest[nch + 1:]

        cid = lax.axis_index("c")
        sid = lax.axis_index("s")
        wid = sid * _NC + cid

        pltpu.sync_copy(src_h.at[pl.ds(wid * epb, epb)], src_v)

        zvec = jnp.zeros((_LANE,), jnp.float32)

        if with_counts:
            lane0 = jnp.where(lax.iota(jnp.int32, _LANE) == 0,
                              jnp.float32(1.0), jnp.float32(0.0))

            @pl.loop(0, _BLK)
            def _(i):
                ones_v[i, pl.ds(0, _LANE)] = lane0
                for j in range(1, _CW // _LANE):
                    ones_v[i, pl.ds(j * _LANE, _LANE)] = zvec

        @pl.loop(0, epb // 8)
        def _(t):
            pltpu.sync_copy(dst_h.at[pl.ds(wid * epb + t * 8, 8)], dst_v)
            pltpu.sync_copy(typ_h.at[pl.ds(wid * epb + t * 8, 8)], typ_v)
            for i in range(8):
                for j in range(_BLK // _LANE):
                    sl = pl.ds(j * _LANE, _LANE)
                    sidx_v[t * 8 + i, sl] = typ_v[i, sl] * n_nodes + dst_v[i, sl]

        nz = rps // _BLK

        def zero_rows():
            @pl.loop(0, _BLK)
            def _(i):
                for j in range(_CW // _LANE):
                    rows_v[i, pl.ds(j * _LANE, _LANE)] = zvec

        if with_counts:
            zero_rows()

            @pl.loop(0, nz)
            def _(t):
                pltpu.sync_copy(rows_v, acc.at[pl.ds(sid * rps + t * _BLK, _BLK)])
            plsc.subcore_barrier()

            @pl.loop(0, epb)
            def _(b):
                pltpu.sync_copy(ones_v, acc.at[sidx_v.at[b]], add=True)
            plsc.subcore_barrier()

            pltpu.sync_copy(acc.at[pl.ds(sid * rps, rps)],
                            cnt_h.at[cid, pl.ds(sid * rps, rps)])
            plsc.subcore_barrier()

        for c in range(nch):
            zero_rows()

            @pl.loop(0, nz)
            def _(t):
                pltpu.sync_copy(rows_v, acc.at[pl.ds(sid * rps + t * _BLK, _BLK)])
            plsc.subcore_barrier()

            @pl.loop(0, epb)
            def _(b):
                pltpu.sync_copy(xs_h[c].at[src_v.at[b]], rows_v)
                pltpu.sync_copy(rows_v, acc.at[sidx_v.at[b]], add=True)
            plsc.subcore_barrier()

            pltpu.sync_copy(acc.at[pl.ds(sid * rps, rps)],
                            sums_h.at[c, cid, pl.ds(sid * rps, rps)])
            plsc.subcore_barrier()

    return pl.kernel(body, out_type=out_type, mesh=mesh, scratch_types=scratch,
                     compiler_params=pltpu.CompilerParams(
                         use_tc_tiling_on_sc=False))


def _make_dense(n_nodes, n_rel, d, emit_chunks):
    nch = d // _CW
    bn = 1000 if n_nodes % 1000 == 0 else n_nodes
    grid = (n_nodes // bn,)

    def body(x_ref, sums_ref, cnt_ref, wrel_ref, wroot_ref, b_ref, g_ref,
             be_ref, *out_refs):
        x = x_ref[...]
        out = jnp.dot(x, wroot_ref[...], preferred_element_type=jnp.float32)
        out = out + b_ref[...]
        for r in range(n_rel):
            cnt = cnt_ref[0, r] + cnt_ref[1, r]
            inv = 1.0 / jnp.maximum(cnt[:, 0:1], 1.0)
            s = jnp.concatenate(
                [sums_ref[c, 0, r] + sums_ref[c, 1, r] for c in range(nch)],
                axis=-1)
            out = out + jnp.dot(s * inv, wrel_ref[r],
                                preferred_element_type=jnp.float32)
        mu = jnp.mean(out, axis=-1, keepdims=True)
        ctr = out - mu
        var = jnp.mean(ctr * ctr, axis=-1, keepdims=True)
        y = ctr * lax.rsqrt(var + 1e-5) * g_ref[...] + be_ref[...]
        y = jnp.where(y > 0, y, jnp.exp(jnp.minimum(y, 0.0)) - 1.0)
        out_refs[0][...] = y
        if emit_chunks:
            for c in range(nch):
                out_refs[1 + c][...] = y[:, c * _CW:(c + 1) * _CW]

    in_specs = [
        pl.BlockSpec((bn, d), lambda i: (i, 0)),
        pl.BlockSpec((nch, _NC, n_rel, bn, _CW), lambda i: (0, 0, 0, i, 0)),
        pl.BlockSpec((_NC, n_rel, bn, _CW), lambda i: (0, 0, i, 0)),
        pl.BlockSpec((n_rel, d, d), lambda i: (0, 0, 0)),
        pl.BlockSpec((d, d), lambda i: (0, 0)),
        pl.BlockSpec((1, d), lambda i: (0, 0)),
        pl.BlockSpec((1, d), lambda i: (0, 0)),
        pl.BlockSpec((1, d), lambda i: (0, 0)),
    ]
    out_shape = [jax.ShapeDtypeStruct((n_nodes, d), jnp.float32)]
    out_specs = [pl.BlockSpec((bn, d), lambda i: (i, 0))]
    if emit_chunks:
        out_shape += [jax.ShapeDtypeStruct((n_nodes, _CW), jnp.float32)
                      for _ in range(nch)]
        out_specs += [pl.BlockSpec((bn, _CW), lambda i: (i, 0))
                      for _ in range(nch)]
    return pl.pallas_call(body, grid=grid, in_specs=in_specs,
                          out_specs=out_specs, out_shape=out_shape)


def kernel(node_embedding, edge_index, edge_type,
           W_rel0, W_root0, b0, g0, be0,
           W_rel1, W_root1, b1, g1, be1):
    x = node_embedding
    n, d = x.shape
    e = edge_index.shape[1]
    r = W_rel0.shape[0]
    nch = d // _CW

    epb = -(-e // (_NT * _BLK * 8)) * 8
    e_pad = epb * _BLK * _NT
    pad = e_pad - e
    src = jnp.concatenate([edge_index[0], jnp.zeros((pad,), jnp.int32)])
    dst = jnp.concatenate([edge_index[1], jnp.full((pad,), n, jnp.int32)])
    typ = jnp.concatenate([edge_type, jnp.full((pad,), r - 1, jnp.int32)])
    src2 = src.reshape(_NT * epb, _BLK)
    dst2 = dst.reshape(_NT * epb, _BLK)
    typ2 = typ.reshape(_NT * epb, _BLK)

    xc = [lax.slice_in_dim(x, c * _CW, (c + 1) * _CW, axis=1)
          for c in range(nch)]

    sc_agg0 = _make_sc_agg(n, r, d, epb, with_counts=True)
    sums0, cnt = sc_agg0(src2, dst2, typ2, *xc)
    sums0 = sums0[:, :, :r * n].reshape(nch, _NC, r, n, _CW)
    cnt = cnt[:, :r * n].reshape(_NC, r, n, _CW)

    dense0 = _make_dense(n, r, d, emit_chunks=True)
    x1, *x1c = dense0(x, sums0, cnt, W_rel0, W_root0,
                      b0.reshape(1, d), g0.reshape(1, d), be0.reshape(1, d))

    sc_agg1 = _make_sc_agg(n, r, d, epb, with_counts=False)
    sums1 = sc_agg1(src2, dst2, typ2, *x1c)
    if isinstance(sums1, (list, tuple)):
        sums1 = sums1[0]
    sums1 = sums1[:, :, :r * n].reshape(nch, _NC, r, n, _CW)

    dense1 = _make_dense(n, r, d, emit_chunks=False)
    out = dense1(x1, sums1, cnt, W_rel1, W_root1,
                 b1.reshape(1, d), g1.reshape(1, d), be1.reshape(1, d))
    if isinstance(out, (list, tuple)):
        out = out[0]
    return out


# speedup vs baseline: 4.6573x; 1.1403x over previous
import jax
import jax.numpy as jnp
from jax import lax
from jax.experimental import pallas as pl
from jax.experimental.pallas import tpu as pltpu
from jax.experimental.pallas import tpu_sc as plsc

_NC = 2
_NS = 16
_NT = _NC * _NS
_LANE = 16
_BLK = 128
_CW = 32


def _make_sc_agg(n_nodes, n_rel, d, epb, with_counts):
    nch = d // _CW
    rps = -(-(n_rel * n_nodes + 1) // (_NS * _BLK)) * _BLK
    acc_rows = rps * _NS

    mesh = plsc.VectorSubcoreMesh(core_axis_name="c", subcore_axis_name="s")
    out_type = [jax.ShapeDtypeStruct((nch, _NC, acc_rows, _CW), jnp.float32)]
    if with_counts:
        out_type.append(jax.ShapeDtypeStruct((_NC, acc_rows, _CW), jnp.float32))

    scratch = [
        pltpu.VMEM((epb, _BLK), jnp.int32),
        pltpu.VMEM((8, _BLK), jnp.int32),
        pltpu.VMEM((8, _BLK), jnp.int32),
        pltpu.VMEM((epb, _BLK), jnp.int32),
        pltpu.VMEM((4, _BLK, _CW), jnp.float32),
        pltpu.VMEM((_BLK, _CW), jnp.float32),
        pltpu.VMEM_SHARED((acc_rows, _CW), jnp.float32),
        pltpu.SemaphoreType.DMA,
        pltpu.SemaphoreType.DMA,
    ]

    def body(src_h, dst_h, typ_h, *rest):
        xs_h = rest[:nch]
        if with_counts:
            sums_h, cnt_h = rest[nch], rest[nch + 1]
            rest = rest[nch + 2:]
        else:
            sums_h = rest[nch]
            rest = rest[nch + 1:]
        (src_v, dst_v, typ_v, sidx_v, ring_v, const_v, acc,
         gsem, ssem) = rest

        cid = lax.axis_index("c")
        sid = lax.axis_index("s")
        wid = sid * _NC + cid

        pltpu.sync_copy(src_h.at[pl.ds(wid * epb, epb)], src_v)

        zvec = jnp.zeros((_LANE,), jnp.float32)

        def fill_const(first_lane_val):
            vecs = [jnp.where(lax.iota(jnp.int32, _LANE) == 0,
                              jnp.float32(first_lane_val), jnp.float32(0.0))
                    ] + [zvec] * (_CW // _LANE - 1)

            @pl.loop(0, _BLK)
            def _(i):
                for j, v in enumerate(vecs):
                    const_v[i, pl.ds(j * _LANE, _LANE)] = v

        @pl.loop(0, epb // 8)
        def _(t):
            pltpu.sync_copy(dst_h.at[pl.ds(wid * epb + t * 8, 8)], dst_v)
            pltpu.sync_copy(typ_h.at[pl.ds(wid * epb + t * 8, 8)], typ_v)
            for i in range(8):
                for j in range(_BLK // _LANE):
                    sl = pl.ds(j * _LANE, _LANE)
                    sidx_v[t * 8 + i, sl] = typ_v[i, sl] * n_nodes + dst_v[i, sl]

        nz = rps // _BLK

        def zero_acc():
            @pl.loop(0, nz)
            def _(t):
                pltpu.async_copy(const_v, acc.at[pl.ds(sid * rps + t * _BLK, _BLK)], gsem)

            @pl.loop(0, nz)
            def _(t):
                pltpu.make_async_copy(const_v, acc.at[pl.ds(sid * rps + t * _BLK, _BLK)], gsem).wait()

        if with_counts:
            fill_const(0.0)
            zero_acc()
            plsc.subcore_barrier()
            fill_const(1.0)

            @pl.loop(0, epb)
            def _(b):
                pltpu.async_copy(const_v, acc.at[sidx_v.at[b]], ssem, add=True)

            @pl.loop(0, epb)
            def _(b):
                pltpu.make_async_copy(const_v, acc.at[sidx_v.at[b]], ssem).wait()
            plsc.subcore_barrier()

            pltpu.sync_copy(acc.at[pl.ds(sid * rps, rps)],
                            cnt_h.at[cid, pl.ds(sid * rps, rps)])
            plsc.subcore_barrier()
            fill_const(0.0)
        else:
            fill_const(0.0)

        ngr = epb // 2

        for c in range(nch):
            xc = xs_h[c]

            def gfire(g, half):
                for q in range(2):
                    b = g * 2 + q
                    pltpu.async_copy(xc.at[src_v.at[b]], ring_v.at[half * 2 + q], gsem)

            def gwait(g, half):
                for q in range(2):
                    b = g * 2 + q
                    pltpu.make_async_copy(xc.at[src_v.at[b]], ring_v.at[half * 2 + q], gsem).wait()

            def sfire(g, half):
                for q in range(2):
                    b = g * 2 + q
                    pltpu.async_copy(ring_v.at[half * 2 + q], acc.at[sidx_v.at[b]], ssem, add=True)

            def swait(g, half):
                for q in range(2):
                    b = g * 2 + q
                    pltpu.make_async_copy(ring_v.at[half * 2 + q], acc.at[sidx_v.at[b]], ssem).wait()

            zero_acc()
            plsc.subcore_barrier()

            gfire(0, 0)

            @pl.loop(0, ngr, step=2)
            def _(g):
                gwait(g, 0)
                sfire(g, 0)

                @pl.when(g > 0)
                def _():
                    swait(g - 1, 1)
                gfire(g + 1, 1)
                gwait(g + 1, 1)
                sfire(g + 1, 1)
                swait(g, 0)

                @pl.when(g + 2 < ngr)
                def _():
                    gfire(g + 2, 0)

            swait(ngr - 1, 1)
            plsc.subcore_barrier()

            pltpu.sync_copy(acc.at[pl.ds(sid * rps, rps)],
                            sums_h.at[c, cid, pl.ds(sid * rps, rps)])
            plsc.subcore_barrier()

    return pl.kernel(body, out_type=out_type, mesh=mesh, scratch_types=scratch,
                     compiler_params=pltpu.CompilerParams(
                         use_tc_tiling_on_sc=False))


def _make_dense(n_nodes, n_rel, d, emit_chunks):
    nch = d // _CW
    bn = 1000 if n_nodes % 1000 == 0 else n_nodes
    grid = (n_nodes // bn,)

    def body(x_ref, sums_ref, cnt_ref, wrel_ref, wroot_ref, b_ref, g_ref,
             be_ref, *out_refs):
        x = x_ref[...]
        out = jnp.dot(x, wroot_ref[...], preferred_element_type=jnp.float32)
        out = out + b_ref[...]
        for r in range(n_rel):
            cnt = cnt_ref[0, r] + cnt_ref[1, r]
            inv = 1.0 / jnp.maximum(cnt[:, 0:1], 1.0)
            s = jnp.concatenate(
                [sums_ref[c, 0, r] + sums_ref[c, 1, r] for c in range(nch)],
                axis=-1)
            out = out + jnp.dot(s * inv, wrel_ref[r],
                                preferred_element_type=jnp.float32)
        mu = jnp.mean(out, axis=-1, keepdims=True)
        ctr = out - mu
        var = jnp.mean(ctr * ctr, axis=-1, keepdims=True)
        y = ctr * lax.rsqrt(var + 1e-5) * g_ref[...] + be_ref[...]
        y = jnp.where(y > 0, y, jnp.exp(jnp.minimum(y, 0.0)) - 1.0)
        out_refs[0][...] = y
        if emit_chunks:
            for c in range(nch):
                out_refs[1 + c][...] = y[:, c * _CW:(c + 1) * _CW]

    in_specs = [
        pl.BlockSpec((bn, d), lambda i: (i, 0)),
        pl.BlockSpec((nch, _NC, n_rel, bn, _CW), lambda i: (0, 0, 0, i, 0)),
        pl.BlockSpec((_NC, n_rel, bn, _CW), lambda i: (0, 0, i, 0)),
        pl.BlockSpec((n_rel, d, d), lambda i: (0, 0, 0)),
        pl.BlockSpec((d, d), lambda i: (0, 0)),
        pl.BlockSpec((1, d), lambda i: (0, 0)),
        pl.BlockSpec((1, d), lambda i: (0, 0)),
        pl.BlockSpec((1, d), lambda i: (0, 0)),
    ]
    out_shape = [jax.ShapeDtypeStruct((n_nodes, d), jnp.float32)]
    out_specs = [pl.BlockSpec((bn, d), lambda i: (i, 0))]
    if emit_chunks:
        out_shape += [jax.ShapeDtypeStruct((n_nodes, _CW), jnp.float32)
                      for _ in range(nch)]
        out_specs += [pl.BlockSpec((bn, _CW), lambda i: (i, 0))
                      for _ in range(nch)]
    return pl.pallas_call(body, grid=grid, in_specs=in_specs,
                          out_specs=out_specs, out_shape=out_shape)


def kernel(node_embedding, edge_index, edge_type,
           W_rel0, W_root0, b0, g0, be0,
           W_rel1, W_root1, b1, g1, be1):
    x = node_embedding
    n, d = x.shape
    e = edge_index.shape[1]
    r = W_rel0.shape[0]
    nch = d // _CW

    epb = -(-e // (_NT * _BLK * 8)) * 8
    e_pad = epb * _BLK * _NT
    pad = e_pad - e
    src = jnp.concatenate([edge_index[0], jnp.zeros((pad,), jnp.int32)])
    dst = jnp.concatenate([edge_index[1], jnp.full((pad,), n, jnp.int32)])
    typ = jnp.concatenate([edge_type, jnp.full((pad,), r - 1, jnp.int32)])
    src2 = src.reshape(_NT * epb, _BLK)
    dst2 = dst.reshape(_NT * epb, _BLK)
    typ2 = typ.reshape(_NT * epb, _BLK)

    xc = [lax.slice_in_dim(x, c * _CW, (c + 1) * _CW, axis=1)
          for c in range(nch)]

    sc_agg0 = _make_sc_agg(n, r, d, epb, with_counts=True)
    sums0, cnt = sc_agg0(src2, dst2, typ2, *xc)
    sums0 = sums0[:, :, :r * n].reshape(nch, _NC, r, n, _CW)
    cnt = cnt[:, :r * n].reshape(_NC, r, n, _CW)

    dense0 = _make_dense(n, r, d, emit_chunks=True)
    x1, *x1c = dense0(x, sums0, cnt, W_rel0, W_root0,
                      b0.reshape(1, d), g0.reshape(1, d), be0.reshape(1, d))

    sc_agg1 = _make_sc_agg(n, r, d, epb, with_counts=False)
    sums1 = sc_agg1(src2, dst2, typ2, *x1c)
    if isinstance(sums1, (list, tuple)):
        sums1 = sums1[0]
    sums1 = sums1[:, :, :r * n].reshape(nch, _NC, r, n, _CW)

    dense1 = _make_dense(n, r, d, emit_chunks=False)
    out = dense1(x1, sums1, cnt, W_rel1, W_root1,
                 b1.reshape(1, d), g1.reshape(1, d), be1.reshape(1, d))
    if isinstance(out, (list, tuple)):
        out = out[0]
    return out


# speedup vs baseline: 5.3226x; 1.1429x over previous
import jax
import jax.numpy as jnp
from jax import lax
from jax.experimental import pallas as pl
from jax.experimental.pallas import tpu as pltpu
from jax.experimental.pallas import tpu_sc as plsc

_NC = 2
_NS = 16
_NT = _NC * _NS
_LANE = 16
_BLK = 128
_CW = 32


def _make_sc_agg(n_nodes, n_rel, d, epb, with_counts):
    nch = d // _CW
    rps = -(-(n_rel * n_nodes + 1) // (_NS * _BLK)) * _BLK
    acc_rows = rps * _NS

    mesh = plsc.VectorSubcoreMesh(core_axis_name="c", subcore_axis_name="s")
    out_type = [jax.ShapeDtypeStruct((nch, _NC, acc_rows, _CW), jnp.float32)]
    if with_counts:
        out_type.append(jax.ShapeDtypeStruct((_NC, acc_rows, _CW), jnp.float32))

    scratch = [
        pltpu.VMEM((epb, _BLK), jnp.int32),
        pltpu.VMEM((8, _BLK), jnp.int32),
        pltpu.VMEM((8, _BLK), jnp.int32),
        pltpu.VMEM((epb, _BLK), jnp.int32),
        pltpu.VMEM((4, _BLK, _CW), jnp.float32),
        pltpu.VMEM((_BLK, _CW), jnp.float32),
        pltpu.VMEM_SHARED((acc_rows, _CW), jnp.float32),
        pltpu.SemaphoreType.DMA,
        pltpu.SemaphoreType.DMA,
    ]

    def body(src_h, dst_h, typ_h, *rest):
        xs_h = rest[:nch]
        if with_counts:
            sums_h, cnt_h = rest[nch], rest[nch + 1]
            rest = rest[nch + 2:]
        else:
            sums_h = rest[nch]
            rest = rest[nch + 1:]
        (src_v, dst_v, typ_v, sidx_v, ring_v, const_v, acc,
         gsem, ssem) = rest

        cid = lax.axis_index("c")
        sid = lax.axis_index("s")
        wid = sid * _NC + cid

        pltpu.sync_copy(src_h.at[pl.ds(wid * epb, epb)], src_v)

        zvec = jnp.zeros((_LANE,), jnp.float32)

        def fill_const(first_lane_val):
            vecs = [jnp.where(lax.iota(jnp.int32, _LANE) == 0,
                              jnp.float32(first_lane_val), jnp.float32(0.0))
                    ] + [zvec] * (_CW // _LANE - 1)

            @pl.loop(0, _BLK)
            def _(i):
                for j, v in enumerate(vecs):
                    const_v[i, pl.ds(j * _LANE, _LANE)] = v

        @pl.loop(0, epb // 8)
        def _(t):
            pltpu.sync_copy(dst_h.at[pl.ds(wid * epb + t * 8, 8)], dst_v)
            pltpu.sync_copy(typ_h.at[pl.ds(wid * epb + t * 8, 8)], typ_v)
            for i in range(8):
                for j in range(_BLK // _LANE):
                    sl = pl.ds(j * _LANE, _LANE)
                    sidx_v[t * 8 + i, sl] = typ_v[i, sl] * n_nodes + dst_v[i, sl]

        nz = rps // _BLK

        def zero_acc():
            @pl.loop(0, nz)
            def _(t):
                pltpu.async_copy(const_v, acc.at[pl.ds(sid * rps + t * _BLK, _BLK)], gsem)

            @pl.loop(0, nz)
            def _(t):
                pltpu.make_async_copy(const_v, acc.at[pl.ds(sid * rps + t * _BLK, _BLK)], gsem).wait()

        if with_counts:
            fill_const(0.0)
            zero_acc()
            plsc.subcore_barrier()
            fill_const(1.0)

            @pl.loop(0, epb)
            def _(b):
                pltpu.async_copy(const_v, acc.at[sidx_v.at[b]], ssem, add=True)

            @pl.loop(0, epb)
            def _(b):
                pltpu.make_async_copy(const_v, acc.at[sidx_v.at[b]], ssem).wait()
            plsc.subcore_barrier()

            pltpu.sync_copy(acc.at[pl.ds(sid * rps, rps)],
                            cnt_h.at[cid, pl.ds(sid * rps, rps)])
            plsc.subcore_barrier()
            fill_const(0.0)
        else:
            fill_const(0.0)

        ngr = epb // 2

        for c in range(nch):
            xc = xs_h[c]

            def gfire(g, half):
                for q in range(2):
                    b = g * 2 + q
                    pltpu.async_copy(xc.at[src_v.at[b]], ring_v.at[half * 2 + q], gsem)

            def gwait(g, half):
                for q in range(2):
                    b = g * 2 + q
                    pltpu.make_async_copy(xc.at[src_v.at[b]], ring_v.at[half * 2 + q], gsem).wait()

            def sfire(g, half):
                for q in range(2):
                    b = g * 2 + q
                    pltpu.async_copy(ring_v.at[half * 2 + q], acc.at[sidx_v.at[b]], ssem, add=True)

            def swait(g, half):
                for q in range(2):
                    b = g * 2 + q
                    pltpu.make_async_copy(ring_v.at[half * 2 + q], acc.at[sidx_v.at[b]], ssem).wait()

            zero_acc()
            plsc.subcore_barrier()

            gfire(0, 0)

            @pl.loop(0, ngr, step=2)
            def _(g):
                gwait(g, 0)
                sfire(g, 0)

                @pl.when(g > 0)
                def _():
                    swait(g - 1, 1)
                gfire(g + 1, 1)
                gwait(g + 1, 1)
                sfire(g + 1, 1)
                swait(g, 0)

                @pl.when(g + 2 < ngr)
                def _():
                    gfire(g + 2, 0)

            swait(ngr - 1, 1)
            plsc.subcore_barrier()

            pltpu.sync_copy(acc.at[pl.ds(sid * rps, rps)],
                            sums_h.at[c, cid, pl.ds(sid * rps, rps)])
            plsc.subcore_barrier()

    return pl.kernel(body, out_type=out_type, mesh=mesh, scratch_types=scratch,
                     compiler_params=pltpu.CompilerParams(
                         use_tc_tiling_on_sc=False))


def _make_dense(n_nodes, n_rel, d, emit_chunks):
    nch = d // _CW
    bn = 1000 if n_nodes % 1000 == 0 else n_nodes
    grid = (n_nodes // bn,)
    nb = n_nodes // bn

    def body(x_ref, *refs):
        sums_refs = refs[:n_rel]
        cnt_refs = refs[n_rel:2 * n_rel]
        wrel_ref, wroot_ref, b_ref, g_ref, be_ref = refs[2 * n_rel:2 * n_rel + 5]
        out_refs = refs[2 * n_rel + 5:]
        x = x_ref[...]
        out = jnp.dot(x, wroot_ref[...], preferred_element_type=jnp.float32)
        out = out + b_ref[...]
        for r in range(n_rel):
            cnt = cnt_refs[r][0] + cnt_refs[r][1]
            inv = 1.0 / jnp.maximum(cnt[:, 0:1], 1.0)
            s = jnp.concatenate(
                [sums_refs[r][c, 0] + sums_refs[r][c, 1] for c in range(nch)],
                axis=-1)
            out = out + jnp.dot(s * inv, wrel_ref[r],
                                preferred_element_type=jnp.float32)
        mu = jnp.mean(out, axis=-1, keepdims=True)
        ctr = out - mu
        var = jnp.mean(ctr * ctr, axis=-1, keepdims=True)
        y = ctr * lax.rsqrt(var + 1e-5) * g_ref[...] + be_ref[...]
        y = jnp.where(y > 0, y, jnp.exp(jnp.minimum(y, 0.0)) - 1.0)
        out_refs[0][...] = y
        if emit_chunks:
            for c in range(nch):
                out_refs[1 + c][...] = y[:, c * _CW:(c + 1) * _CW]

    in_specs = [pl.BlockSpec((bn, d), lambda i: (i, 0))]
    in_specs += [pl.BlockSpec((nch, _NC, bn, _CW),
                              lambda i, r=r: (0, 0, r * nb + i, 0))
                 for r in range(n_rel)]
    in_specs += [pl.BlockSpec((_NC, bn, _CW),
                              lambda i, r=r: (0, r * nb + i, 0))
                 for r in range(n_rel)]
    in_specs += [
        pl.BlockSpec((n_rel, d, d), lambda i: (0, 0, 0)),
        pl.BlockSpec((d, d), lambda i: (0, 0)),
        pl.BlockSpec((1, d), lambda i: (0, 0)),
        pl.BlockSpec((1, d), lambda i: (0, 0)),
        pl.BlockSpec((1, d), lambda i: (0, 0)),
    ]
    out_shape = [jax.ShapeDtypeStruct((n_nodes, d), jnp.float32)]
    out_specs = [pl.BlockSpec((bn, d), lambda i: (i, 0))]
    if emit_chunks:
        out_shape += [jax.ShapeDtypeStruct((n_nodes, _CW), jnp.float32)
                      for _ in range(nch)]
        out_specs += [pl.BlockSpec((bn, _CW), lambda i: (i, 0))
                      for _ in range(nch)]
    return pl.pallas_call(body, grid=grid, in_specs=in_specs,
                          out_specs=out_specs, out_shape=out_shape)


def kernel(node_embedding, edge_index, edge_type,
           W_rel0, W_root0, b0, g0, be0,
           W_rel1, W_root1, b1, g1, be1):
    x = node_embedding
    n, d = x.shape
    e = edge_index.shape[1]
    r = W_rel0.shape[0]
    nch = d // _CW

    epb = -(-e // (_NT * _BLK * 8)) * 8
    e_pad = epb * _BLK * _NT
    pad = e_pad - e
    src = jnp.concatenate([edge_index[0], jnp.zeros((pad,), jnp.int32)])
    dst = jnp.concatenate([edge_index[1], jnp.full((pad,), n, jnp.int32)])
    typ = jnp.concatenate([edge_type, jnp.full((pad,), r - 1, jnp.int32)])
    src2 = src.reshape(_NT * epb, _BLK)
    dst2 = dst.reshape(_NT * epb, _BLK)
    typ2 = typ.reshape(_NT * epb, _BLK)

    xc = [lax.slice_in_dim(x, c * _CW, (c + 1) * _CW, axis=1)
          for c in range(nch)]

    sc_agg0 = _make_sc_agg(n, r, d, epb, with_counts=True)
    sums0, cnt = sc_agg0(src2, dst2, typ2, *xc)

    dense0 = _make_dense(n, r, d, emit_chunks=True)
    x1, *x1c = dense0(x, *([sums0] * r), *([cnt] * r), W_rel0, W_root0,
                      b0.reshape(1, d), g0.reshape(1, d), be0.reshape(1, d))

    sc_agg1 = _make_sc_agg(n, r, d, epb, with_counts=False)
    sums1 = sc_agg1(src2, dst2, typ2, *x1c)
    if isinstance(sums1, (list, tuple)):
        sums1 = sums1[0]

    dense1 = _make_dense(n, r, d, emit_chunks=False)
    out = dense1(x1, *([sums1] * r), *([cnt] * r), W_rel1, W_root1,
                 b1.reshape(1, d), g1.reshape(1, d), be1.reshape(1, d))
    if isinstance(out, (list, tuple)):
        out = out[0]
    return out


# speedup vs baseline: 6.1689x; 1.1590x over previous
import jax
import jax.numpy as jnp
from jax import lax
from jax.experimental import pallas as pl
from jax.experimental.pallas import tpu as pltpu
from jax.experimental.pallas import tpu_sc as plsc

_NC = 2
_NS = 16
_NT = _NC * _NS
_LANE = 16
_BLK = 128
_CW = 32


def _make_sc_agg(n_nodes, n_rel, d, epb, with_counts):
    nch = d // _CW
    rps = -(-(n_rel * n_nodes + 1) // (_NS * _BLK)) * _BLK
    acc_rows = rps * _NS

    mesh = plsc.VectorSubcoreMesh(core_axis_name="c", subcore_axis_name="s")
    out_type = [jax.ShapeDtypeStruct((_NC, acc_rows, d), jnp.float32)]
    if with_counts:
        out_type.append(jax.ShapeDtypeStruct((_NC, acc_rows, _CW), jnp.float32))

    scratch = [
        pltpu.VMEM((epb, _BLK), jnp.int32),
        pltpu.VMEM((8, _BLK), jnp.int32),
        pltpu.VMEM((8, _BLK), jnp.int32),
        pltpu.VMEM((epb, _BLK), jnp.int32),
        pltpu.VMEM((4, _BLK, _CW), jnp.float32),
        pltpu.VMEM((_BLK, _CW), jnp.float32),
        pltpu.VMEM_SHARED((acc_rows, _CW), jnp.float32),
        pltpu.SemaphoreType.DMA,
        pltpu.SemaphoreType.DMA,
    ]

    def body(src_h, dst_h, typ_h, *rest):
        xs_h = rest[:nch]
        if with_counts:
            sums_h, cnt_h = rest[nch], rest[nch + 1]
            rest = rest[nch + 2:]
        else:
            sums_h = rest[nch]
            rest = rest[nch + 1:]
        (src_v, dst_v, typ_v, sidx_v, ring_v, const_v, acc,
         gsem, ssem) = rest

        cid = lax.axis_index("c")
        sid = lax.axis_index("s")
        wid = sid * _NC + cid

        pltpu.sync_copy(src_h.at[pl.ds(wid * epb, epb)], src_v)

        zvec = jnp.zeros((_LANE,), jnp.float32)

        def fill_const(first_lane_val):
            vecs = [jnp.where(lax.iota(jnp.int32, _LANE) == 0,
                              jnp.float32(first_lane_val), jnp.float32(0.0))
                    ] + [zvec] * (_CW // _LANE - 1)

            @pl.loop(0, _BLK)
            def _(i):
                for j, v in enumerate(vecs):
                    const_v[i, pl.ds(j * _LANE, _LANE)] = v

        @pl.loop(0, epb // 8)
        def _(t):
            pltpu.sync_copy(dst_h.at[pl.ds(wid * epb + t * 8, 8)], dst_v)
            pltpu.sync_copy(typ_h.at[pl.ds(wid * epb + t * 8, 8)], typ_v)
            for i in range(8):
                for j in range(_BLK // _LANE):
                    sl = pl.ds(j * _LANE, _LANE)
                    sidx_v[t * 8 + i, sl] = typ_v[i, sl] * n_nodes + dst_v[i, sl]

        nz = rps // _BLK

        def zero_acc():
            @pl.loop(0, nz)
            def _(t):
                pltpu.async_copy(const_v, acc.at[pl.ds(sid * rps + t * _BLK, _BLK)], gsem)

            @pl.loop(0, nz)
            def _(t):
                pltpu.make_async_copy(const_v, acc.at[pl.ds(sid * rps + t * _BLK, _BLK)], gsem).wait()

        if with_counts:
            fill_const(0.0)
            zero_acc()
            plsc.subcore_barrier()
            fill_const(1.0)

            @pl.loop(0, epb)
            def _(b):
                pltpu.async_copy(const_v, acc.at[sidx_v.at[b]], ssem, add=True)

            @pl.loop(0, epb)
            def _(b):
                pltpu.make_async_copy(const_v, acc.at[sidx_v.at[b]], ssem).wait()
            plsc.subcore_barrier()

            pltpu.sync_copy(acc.at[pl.ds(sid * rps, rps)],
                            cnt_h.at[cid, pl.ds(sid * rps, rps)])
            plsc.subcore_barrier()
            fill_const(0.0)
        else:
            fill_const(0.0)

        ngr = epb // 2

        for c in range(nch):
            xc = xs_h[c]

            def gfire(g, half):
                for q in range(2):
                    b = g * 2 + q
                    pltpu.async_copy(xc.at[src_v.at[b]], ring_v.at[half * 2 + q], gsem)

            def gwait(g, half):
                for q in range(2):
                    b = g * 2 + q
                    pltpu.make_async_copy(xc.at[src_v.at[b]], ring_v.at[half * 2 + q], gsem).wait()

            def sfire(g, half):
                for q in range(2):
                    b = g * 2 + q
                    pltpu.async_copy(ring_v.at[half * 2 + q], acc.at[sidx_v.at[b]], ssem, add=True)

            def swait(g, half):
                for q in range(2):
                    b = g * 2 + q
                    pltpu.make_async_copy(ring_v.at[half * 2 + q], acc.at[sidx_v.at[b]], ssem).wait()

            zero_acc()
            plsc.subcore_barrier()

            gfire(0, 0)

            @pl.loop(0, ngr, step=2)
            def _(g):
                gwait(g, 0)
                sfire(g, 0)

                @pl.when(g > 0)
                def _():
                    swait(g - 1, 1)
                gfire(g + 1, 1)
                gwait(g + 1, 1)
                sfire(g + 1, 1)
                swait(g, 0)

                @pl.when(g + 2 < ngr)
                def _():
                    gfire(g + 2, 0)

            swait(ngr - 1, 1)
            plsc.subcore_barrier()

            pltpu.sync_copy(acc.at[pl.ds(sid * rps, rps)],
                            sums_h.at[cid, pl.ds(sid * rps, rps),
                                      pl.ds(c * _CW, _CW)])
            plsc.subcore_barrier()

    return pl.kernel(body, out_type=out_type, mesh=mesh, scratch_types=scratch,
                     compiler_params=pltpu.CompilerParams(
                         use_tc_tiling_on_sc=False))


def _make_dense(n_nodes, n_rel, d, emit_chunks):
    nch = d // _CW
    bn = 1000 if n_nodes % 1000 == 0 else n_nodes
    grid = (n_nodes // bn,)
    nb = n_nodes // bn

    def body(x_ref, *refs):
        sums_refs = refs[:n_rel]
        cnt_refs = refs[n_rel:2 * n_rel]
        wrel_ref, wroot_ref, b_ref, g_ref, be_ref = refs[2 * n_rel:2 * n_rel + 5]
        out_refs = refs[2 * n_rel + 5:]
        x = x_ref[...]
        out = jnp.dot(x, wroot_ref[...], preferred_element_type=jnp.float32)
        out = out + b_ref[...]
        for r in range(n_rel):
            cnt = cnt_refs[r][0] + cnt_refs[r][1]
            inv = 1.0 / jnp.maximum(cnt[:, 0:1], 1.0)
            s = sums_refs[r][0] + sums_refs[r][1]
            out = out + jnp.dot(s * inv, wrel_ref[r],
                                preferred_element_type=jnp.float32)
        mu = jnp.mean(out, axis=-1, keepdims=True)
        ctr = out - mu
        var = jnp.mean(ctr * ctr, axis=-1, keepdims=True)
        y = ctr * lax.rsqrt(var + 1e-5) * g_ref[...] + be_ref[...]
        y = jnp.where(y > 0, y, jnp.exp(jnp.minimum(y, 0.0)) - 1.0)
        out_refs[0][...] = y
        if emit_chunks:
            for c in range(nch):
                out_refs[1 + c][...] = y[:, c * _CW:(c + 1) * _CW]

    in_specs = [pl.BlockSpec((bn, d), lambda i: (i, 0))]
    in_specs += [pl.BlockSpec((_NC, bn, d),
                              lambda i, r=r: (0, r * nb + i, 0))
                 for r in range(n_rel)]
    in_specs += [pl.BlockSpec((_NC, bn, _CW),
                              lambda i, r=r: (0, r * nb + i, 0))
                 for r in range(n_rel)]
    in_specs += [
        pl.BlockSpec((n_rel, d, d), lambda i: (0, 0, 0)),
        pl.BlockSpec((d, d), lambda i: (0, 0)),
        pl.BlockSpec((1, d), lambda i: (0, 0)),
        pl.BlockSpec((1, d), lambda i: (0, 0)),
        pl.BlockSpec((1, d), lambda i: (0, 0)),
    ]
    out_shape = [jax.ShapeDtypeStruct((n_nodes, d), jnp.float32)]
    out_specs = [pl.BlockSpec((bn, d), lambda i: (i, 0))]
    if emit_chunks:
        out_shape += [jax.ShapeDtypeStruct((n_nodes, _CW), jnp.float32)
                      for _ in range(nch)]
        out_specs += [pl.BlockSpec((bn, _CW), lambda i: (i, 0))
                      for _ in range(nch)]
    return pl.pallas_call(body, grid=grid, in_specs=in_specs,
                          out_specs=out_specs, out_shape=out_shape)


def kernel(node_embedding, edge_index, edge_type,
           W_rel0, W_root0, b0, g0, be0,
           W_rel1, W_root1, b1, g1, be1):
    x = node_embedding
    n, d = x.shape
    e = edge_index.shape[1]
    r = W_rel0.shape[0]
    nch = d // _CW

    epb = -(-e // (_NT * _BLK * 8)) * 8
    e_pad = epb * _BLK * _NT
    pad = e_pad - e
    src = jnp.concatenate([edge_index[0], jnp.zeros((pad,), jnp.int32)])
    dst = jnp.concatenate([edge_index[1], jnp.full((pad,), n, jnp.int32)])
    typ = jnp.concatenate([edge_type, jnp.full((pad,), r - 1, jnp.int32)])
    src2 = src.reshape(_NT * epb, _BLK)
    dst2 = dst.reshape(_NT * epb, _BLK)
    typ2 = typ.reshape(_NT * epb, _BLK)

    xc = [lax.slice_in_dim(x, c * _CW, (c + 1) * _CW, axis=1)
          for c in range(nch)]

    sc_agg0 = _make_sc_agg(n, r, d, epb, with_counts=True)
    sums0, cnt = sc_agg0(src2, dst2, typ2, *xc)

    dense0 = _make_dense(n, r, d, emit_chunks=True)
    x1, *x1c = dense0(x, *([sums0] * r), *([cnt] * r), W_rel0, W_root0,
                      b0.reshape(1, d), g0.reshape(1, d), be0.reshape(1, d))

    sc_agg1 = _make_sc_agg(n, r, d, epb, with_counts=False)
    sums1 = sc_agg1(src2, dst2, typ2, *x1c)
    if isinstance(sums1, (list, tuple)):
        sums1 = sums1[0]

    dense1 = _make_dense(n, r, d, emit_chunks=False)
    out = dense1(x1, *([sums1] * r), *([cnt] * r), W_rel1, W_root1,
                 b1.reshape(1, d), g1.reshape(1, d), be1.reshape(1, d))
    if isinstance(out, (list, tuple)):
        out = out[0]
    return out


# speedup vs baseline: 7.3508x; 1.1916x over previous
import jax
import jax.numpy as jnp
from jax import lax
from jax.experimental import pallas as pl
from jax.experimental.pallas import tpu as pltpu
from jax.experimental.pallas import tpu_sc as plsc

_NC = 2
_NS = 16
_NT = _NC * _NS
_LANE = 16
_BLK = 128
_CW = 32


def _make_sc_agg(n_nodes, n_rel, d, epb0, epb1, with_counts):
    nch = d // _CW
    tot = epb0 + epb1
    rps = -(-(n_rel * n_nodes + 1) // (_NS * _BLK)) * _BLK
    acc_rows = rps * _NS

    mesh = plsc.VectorSubcoreMesh(core_axis_name="c", subcore_axis_name="s")
    out_type = [jax.ShapeDtypeStruct((_NC, acc_rows, d), jnp.float32)]
    if with_counts:
        out_type.append(jax.ShapeDtypeStruct((_NC, acc_rows, _CW), jnp.float32))

    scratch = [
        pltpu.VMEM((epb0, _BLK), jnp.int32),
        pltpu.VMEM((8, _BLK), jnp.int32),
        pltpu.VMEM((8, _BLK), jnp.int32),
        pltpu.VMEM((epb0, _BLK), jnp.int32),
        pltpu.VMEM((4, _BLK, _CW), jnp.float32),
        pltpu.VMEM_SHARED((acc_rows, _CW), jnp.float32),
        pltpu.SemaphoreType.DMA,
        pltpu.SemaphoreType.DMA,
    ]

    def body(src_h, dst_h, typ_h, *rest):
        xs_h = rest[:nch]
        if with_counts:
            sums_h, cnt_h = rest[nch], rest[nch + 1]
            rest = rest[nch + 2:]
        else:
            sums_h = rest[nch]
            rest = rest[nch + 1:]
        (src_v, dst_v, typ_v, sidx_v, ring_v, acc,
         gsem, ssem) = rest

        cid = lax.axis_index("c")
        sid = lax.axis_index("s")
        base = sid * tot + cid * epb0
        epb = jnp.where(cid == 0, epb0, epb1)

        pltpu.sync_copy(src_h.at[pl.ds(base, epb0)], src_v)

        zvec = jnp.zeros((_LANE,), jnp.float32)
        const_v = ring_v.at[0]

        def fill_const(first_lane_val):
            vecs = [jnp.where(lax.iota(jnp.int32, _LANE) == 0,
                              jnp.float32(first_lane_val), jnp.float32(0.0))
                    ] + [zvec] * (_CW // _LANE - 1)

            @pl.loop(0, _BLK)
            def _(i):
                for j, v in enumerate(vecs):
                    const_v[i, pl.ds(j * _LANE, _LANE)] = v

        @pl.loop(0, epb // 8)
        def _(t):
            pltpu.sync_copy(dst_h.at[pl.ds(base + t * 8, 8)], dst_v)
            pltpu.sync_copy(typ_h.at[pl.ds(base + t * 8, 8)], typ_v)
            for i in range(8):
                for j in range(_BLK // _LANE):
                    sl = pl.ds(j * _LANE, _LANE)
                    sidx_v[t * 8 + i, sl] = typ_v[i, sl] * n_nodes + dst_v[i, sl]

        nz = rps // _BLK

        def zero_acc():
            @pl.loop(0, nz)
            def _(t):
                pltpu.async_copy(const_v, acc.at[pl.ds(sid * rps + t * _BLK, _BLK)], gsem)

            @pl.loop(0, nz)
            def _(t):
                pltpu.make_async_copy(const_v, acc.at[pl.ds(sid * rps + t * _BLK, _BLK)], gsem).wait()

        if with_counts:
            fill_const(0.0)
            zero_acc()
            plsc.subcore_barrier()
            fill_const(1.0)

            @pl.loop(0, epb)
            def _(b):
                pltpu.async_copy(const_v, acc.at[sidx_v.at[b]], ssem, add=True)

            @pl.loop(0, epb)
            def _(b):
                pltpu.make_async_copy(const_v, acc.at[sidx_v.at[b]], ssem).wait()
            plsc.subcore_barrier()

            pltpu.sync_copy(acc.at[pl.ds(sid * rps, rps)],
                            cnt_h.at[cid, pl.ds(sid * rps, rps)])
            plsc.subcore_barrier()

        ngr = epb // 2

        for c in range(nch):
            xc = xs_h[c]

            def gfire(g, half):
                for q in range(2):
                    b = g * 2 + q
                    pltpu.async_copy(xc.at[src_v.at[b]], ring_v.at[half * 2 + q], gsem)

            def gwait(g, half):
                for q in range(2):
                    b = g * 2 + q
                    pltpu.make_async_copy(xc.at[src_v.at[b]], ring_v.at[half * 2 + q], gsem).wait()

            def sfire(g, half):
                for q in range(2):
                    b = g * 2 + q
                    pltpu.async_copy(ring_v.at[half * 2 + q], acc.at[sidx_v.at[b]], ssem, add=True)

            def swait(g, half):
                for q in range(2):
                    b = g * 2 + q
                    pltpu.make_async_copy(ring_v.at[half * 2 + q], acc.at[sidx_v.at[b]], ssem).wait()

            fill_const(0.0)
            zero_acc()
            plsc.subcore_barrier()

            gfire(0, 0)

            @pl.loop(0, ngr, step=2)
            def _(g):
                gwait(g, 0)
                sfire(g, 0)

                @pl.when(g > 0)
                def _():
                    swait(g - 1, 1)
                gfire(g + 1, 1)
                gwait(g + 1, 1)
                sfire(g + 1, 1)
                swait(g, 0)

                @pl.when(g + 2 < ngr)
                def _():
                    gfire(g + 2, 0)

            swait(ngr - 1, 1)
            plsc.subcore_barrier()

            pltpu.sync_copy(acc.at[pl.ds(sid * rps, rps)],
                            sums_h.at[cid, pl.ds(sid * rps, rps),
                                      pl.ds(c * _CW, _CW)])
            plsc.subcore_barrier()

    return pl.kernel(body, out_type=out_type, mesh=mesh, scratch_types=scratch,
                     compiler_params=pltpu.CompilerParams(
                         use_tc_tiling_on_sc=False))


def _make_dense(n_nodes, n_rel, d, emit_chunks):
    nch = d // _CW
    bn = 1000 if n_nodes % 1000 == 0 else n_nodes
    grid = (n_nodes // bn,)
    nb = n_nodes // bn

    def body(x_ref, *refs):
        sums_refs = refs[:n_rel]
        cnt_refs = refs[n_rel:2 * n_rel]
        wrel_ref, wroot_ref, b_ref, g_ref, be_ref = refs[2 * n_rel:2 * n_rel + 5]
        out_refs = refs[2 * n_rel + 5:]
        x = x_ref[...]
        out = jnp.dot(x, wroot_ref[...], preferred_element_type=jnp.float32)
        out = out + b_ref[...]
        for r in range(n_rel):
            cnt = cnt_refs[r][0] + cnt_refs[r][1]
            inv = 1.0 / jnp.maximum(cnt[:, 0:1], 1.0)
            s = sums_refs[r][0] + sums_refs[r][1]
            out = out + jnp.dot(s * inv, wrel_ref[r],
                                preferred_element_type=jnp.float32)
        mu = jnp.mean(out, axis=-1, keepdims=True)
        ctr = out - mu
        var = jnp.mean(ctr * ctr, axis=-1, keepdims=True)
        y = ctr * lax.rsqrt(var + 1e-5) * g_ref[...] + be_ref[...]
        y = jnp.where(y > 0, y, jnp.exp(jnp.minimum(y, 0.0)) - 1.0)
        out_refs[0][...] = y
        if emit_chunks:
            for c in range(nch):
                out_refs[1 + c][...] = y[:, c * _CW:(c + 1) * _CW]

    in_specs = [pl.BlockSpec((bn, d), lambda i: (i, 0))]
    in_specs += [pl.BlockSpec((_NC, bn, d),
                              lambda i, r=r: (0, r * nb + i, 0))
                 for r in range(n_rel)]
    in_specs += [pl.BlockSpec((_NC, bn, _CW),
                              lambda i, r=r: (0, r * nb + i, 0))
                 for r in range(n_rel)]
    in_specs += [
        pl.BlockSpec((n_rel, d, d), lambda i: (0, 0, 0)),
        pl.BlockSpec((d, d), lambda i: (0, 0)),
        pl.BlockSpec((1, d), lambda i: (0, 0)),
        pl.BlockSpec((1, d), lambda i: (0, 0)),
        pl.BlockSpec((1, d), lambda i: (0, 0)),
    ]
    out_shape = [jax.ShapeDtypeStruct((n_nodes, d), jnp.float32)]
    out_specs = [pl.BlockSpec((bn, d), lambda i: (i, 0))]
    if emit_chunks:
        out_shape += [jax.ShapeDtypeStruct((n_nodes, _CW), jnp.float32)
                      for _ in range(nch)]
        out_specs += [pl.BlockSpec((bn, _CW), lambda i: (i, 0))
                      for _ in range(nch)]
    return pl.pallas_call(body, grid=grid, in_specs=in_specs,
                          out_specs=out_specs, out_shape=out_shape)


def kernel(node_embedding, edge_index, edge_type,
           W_rel0, W_root0, b0, g0, be0,
           W_rel1, W_root1, b1, g1, be1):
    x = node_embedding
    n, d = x.shape
    e = edge_index.shape[1]
    r = W_rel0.shape[0]
    nch = d // _CW

    tot = -(-e // (_NS * _BLK * 16)) * 16
    epb0 = max(8, int(round(0.7 * tot / 8)) * 8)
    epb1 = tot - epb0
    nrows = _NS * tot + (epb0 - epb1)
    pad = nrows * _BLK - e
    src = jnp.concatenate([edge_index[0], jnp.zeros((pad,), jnp.int32)])
    dst = jnp.concatenate([edge_index[1], jnp.full((pad,), n, jnp.int32)])
    typ = jnp.concatenate([edge_type, jnp.full((pad,), r - 1, jnp.int32)])
    src2 = src.reshape(nrows, _BLK)
    dst2 = dst.reshape(nrows, _BLK)
    typ2 = typ.reshape(nrows, _BLK)

    xc = [lax.slice_in_dim(x, c * _CW, (c + 1) * _CW, axis=1)
          for c in range(nch)]

    sc_agg0 = _make_sc_agg(n, r, d, epb0, epb1, with_counts=True)
    sums0, cnt = sc_agg0(src2, dst2, typ2, *xc)

    dense0 = _make_dense(n, r, d, emit_chunks=True)
    x1, *x1c = dense0(x, *([sums0] * r), *([cnt] * r), W_rel0, W_root0,
                      b0.reshape(1, d), g0.reshape(1, d), be0.reshape(1, d))

    sc_agg1 = _make_sc_agg(n, r, d, epb0, epb1, with_counts=False)
    sums1 = sc_agg1(src2, dst2, typ2, *x1c)
    if isinstance(sums1, (list, tuple)):
        sums1 = sums1[0]

    dense1 = _make_dense(n, r, d, emit_chunks=False)
    out = dense1(x1, *([sums1] * r), *([cnt] * r), W_rel1, W_root1,
                 b1.reshape(1, d), g1.reshape(1, d), be1.reshape(1, d))
    if isinstance(out, (list, tuple)):
        out = out[0]
    return out


# speedup vs baseline: 7.7198x; 1.0502x over previous
import jax
import jax.numpy as jnp
from jax import lax
from jax.experimental import pallas as pl
from jax.experimental.pallas import tpu as pltpu
from jax.experimental.pallas import tpu_sc as plsc

_NC = 2
_NS = 16
_NT = _NC * _NS
_LANE = 16
_BLK = 64
_CW = 32


def _make_sc_agg(n_nodes, n_rel, d, epb0, epb1, with_counts):
    nch = d // _CW
    tot = epb0 + epb1
    rps = -(-(n_rel * n_nodes + 1) // (_NS * _BLK)) * _BLK
    acc_rows = rps * _NS

    mesh = plsc.VectorSubcoreMesh(core_axis_name="c", subcore_axis_name="s")
    out_type = [jax.ShapeDtypeStruct((_NC, acc_rows, d), jnp.float32)]
    if with_counts:
        out_type.append(jax.ShapeDtypeStruct((_NC, acc_rows, d), jnp.float32))

    scratch = [
        pltpu.VMEM((epb0, _BLK), jnp.int32),
        pltpu.VMEM((8, _BLK), jnp.int32),
        pltpu.VMEM((8, _BLK), jnp.int32),
        pltpu.VMEM((epb0, _BLK), jnp.int32),
        pltpu.VMEM((8, _BLK, _CW), jnp.float32),
        pltpu.VMEM_SHARED((acc_rows, _CW), jnp.float32),
        pltpu.SemaphoreType.DMA,
        pltpu.SemaphoreType.DMA,
    ]

    def body(src_h, dst_h, typ_h, *rest):
        xs_h = rest[:nch]
        if with_counts:
            sums_h, cnt_h = rest[nch], rest[nch + 1]
            rest = rest[nch + 2:]
        else:
            sums_h = rest[nch]
            rest = rest[nch + 1:]
        (src_v, dst_v, typ_v, sidx_v, ring_v, acc,
         gsem, ssem) = rest

        cid = lax.axis_index("c")
        sid = lax.axis_index("s")
        base = sid * tot + cid * epb0
        epb = jnp.where(cid == 0, epb0, epb1)

        pltpu.sync_copy(src_h.at[pl.ds(base, epb0)], src_v)

        zvec = jnp.zeros((_LANE,), jnp.float32)
        const_v = ring_v.at[0]

        def fill_const(first_lane_val):
            vecs = [jnp.where(lax.iota(jnp.int32, _LANE) == 0,
                              jnp.float32(first_lane_val), jnp.float32(0.0))
                    ] + [zvec] * (_CW // _LANE - 1)

            @pl.loop(0, _BLK)
            def _(i):
                for j, v in enumerate(vecs):
                    const_v[i, pl.ds(j * _LANE, _LANE)] = v

        @pl.loop(0, epb // 8)
        def _(t):
            pltpu.sync_copy(dst_h.at[pl.ds(base + t * 8, 8)], dst_v)
            pltpu.sync_copy(typ_h.at[pl.ds(base + t * 8, 8)], typ_v)
            for i in range(8):
                for j in range(_BLK // _LANE):
                    sl = pl.ds(j * _LANE, _LANE)
                    sidx_v[t * 8 + i, sl] = typ_v[i, sl] * n_nodes + dst_v[i, sl]

        nz = rps // _BLK

        def zero_acc():
            @pl.loop(0, nz)
            def _(t):
                pltpu.async_copy(const_v, acc.at[pl.ds(sid * rps + t * _BLK, _BLK)], gsem)

            @pl.loop(0, nz)
            def _(t):
                pltpu.make_async_copy(const_v, acc.at[pl.ds(sid * rps + t * _BLK, _BLK)], gsem).wait()

        if with_counts:
            fill_const(0.0)
            zero_acc()
            plsc.subcore_barrier()
            fill_const(1.0)

            @pl.loop(0, epb)
            def _(b):
                pltpu.async_copy(const_v, acc.at[sidx_v.at[b]], ssem, add=True)

            @pl.loop(0, epb)
            def _(b):
                pltpu.make_async_copy(const_v, acc.at[sidx_v.at[b]], ssem).wait()
            plsc.subcore_barrier()

            pltpu.sync_copy(acc.at[pl.ds(sid * rps, rps)],
                            cnt_h.at[cid, pl.ds(sid * rps, rps),
                                     pl.ds(0, _CW)])
            plsc.subcore_barrier()

        ngr = epb // 4

        for c in range(nch):
            xc = xs_h[c]

            def gfire(g, half):
                for q in range(4):
                    b = g * 4 + q
                    pltpu.async_copy(xc.at[src_v.at[b]], ring_v.at[half * 4 + q], gsem)

            def gwait(g, half):
                for q in range(4):
                    b = g * 4 + q
                    pltpu.make_async_copy(xc.at[src_v.at[b]], ring_v.at[half * 4 + q], gsem).wait()

            def sfire(g, half):
                for q in range(4):
                    b = g * 4 + q
                    pltpu.async_copy(ring_v.at[half * 4 + q], acc.at[sidx_v.at[b]], ssem, add=True)

            def swait(g, half):
                for q in range(4):
                    b = g * 4 + q
                    pltpu.make_async_copy(ring_v.at[half * 4 + q], acc.at[sidx_v.at[b]], ssem).wait()

            fill_const(0.0)
            zero_acc()
            plsc.subcore_barrier()

            gfire(0, 0)

            @pl.loop(0, ngr, step=2)
            def _(g):
                gwait(g, 0)
                sfire(g, 0)

                @pl.when(g > 0)
                def _():
                    swait(g - 1, 1)
                gfire(g + 1, 1)
                gwait(g + 1, 1)
                sfire(g + 1, 1)
                swait(g, 0)

                @pl.when(g + 2 < ngr)
                def _():
                    gfire(g + 2, 0)

            swait(ngr - 1, 1)
            plsc.subcore_barrier()

            pltpu.sync_copy(acc.at[pl.ds(sid * rps, rps)],
                            sums_h.at[cid, pl.ds(sid * rps, rps),
                                      pl.ds(c * _CW, _CW)])
            plsc.subcore_barrier()

    return pl.kernel(body, out_type=out_type, mesh=mesh, scratch_types=scratch,
                     compiler_params=pltpu.CompilerParams(
                         use_tc_tiling_on_sc=False))


def _make_dense(n_nodes, n_rel, d, emit_chunks):
    nch = d // _CW
    bn = 1000 if n_nodes % 1000 == 0 else n_nodes
    grid = (n_nodes // bn,)
    nb = n_nodes // bn

    def body(x_ref, *refs):
        sums_refs = refs[:n_rel]
        cnt_refs = refs[n_rel:2 * n_rel]
        wrel_ref, wroot_ref, b_ref, g_ref, be_ref = refs[2 * n_rel:2 * n_rel + 5]
        out_refs = refs[2 * n_rel + 5:]
        x = x_ref[...]
        out = jnp.dot(x, wroot_ref[...], preferred_element_type=jnp.float32)
        out = out + b_ref[...]
        for r in range(n_rel):
            cnt = cnt_refs[r][0] + cnt_refs[r][1]
            inv = 1.0 / jnp.maximum(cnt[:, 0:1], 1.0)
            s = sums_refs[r][0] + sums_refs[r][1]
            out = out + jnp.dot(s * inv, wrel_ref[r],
                                preferred_element_type=jnp.float32)
        mu = jnp.mean(out, axis=-1, keepdims=True)
        ctr = out - mu
        var = jnp.mean(ctr * ctr, axis=-1, keepdims=True)
        y = ctr * lax.rsqrt(var + 1e-5) * g_ref[...] + be_ref[...]
        y = jnp.where(y > 0, y, jnp.exp(jnp.minimum(y, 0.0)) - 1.0)
        out_refs[0][...] = y
        if emit_chunks:
            for c in range(nch):
                out_refs[1 + c][...] = y[:, c * _CW:(c + 1) * _CW]

    in_specs = [pl.BlockSpec((bn, d), lambda i: (i, 0))]
    in_specs += [pl.BlockSpec((_NC, bn, d),
                              lambda i, r=r: (0, r * nb + i, 0))
                 for r in range(n_rel)]
    in_specs += [pl.BlockSpec((_NC, bn, d),
                              lambda i, r=r: (0, r * nb + i, 0))
                 for r in range(n_rel)]
    in_specs += [
        pl.BlockSpec((n_rel, d, d), lambda i: (0, 0, 0)),
        pl.BlockSpec((d, d), lambda i: (0, 0)),
        pl.BlockSpec((1, d), lambda i: (0, 0)),
        pl.BlockSpec((1, d), lambda i: (0, 0)),
        pl.BlockSpec((1, d), lambda i: (0, 0)),
    ]
    out_shape = [jax.ShapeDtypeStruct((n_nodes, d), jnp.float32)]
    out_specs = [pl.BlockSpec((bn, d), lambda i: (i, 0))]
    if emit_chunks:
        out_shape += [jax.ShapeDtypeStruct((n_nodes, _CW), jnp.float32)
                      for _ in range(nch)]
        out_specs += [pl.BlockSpec((bn, _CW), lambda i: (i, 0))
                      for _ in range(nch)]
    return pl.pallas_call(body, grid=grid, in_specs=in_specs,
                          out_specs=out_specs, out_shape=out_shape)


def kernel(node_embedding, edge_index, edge_type,
           W_rel0, W_root0, b0, g0, be0,
           W_rel1, W_root1, b1, g1, be1):
    x = node_embedding
    n, d = x.shape
    e = edge_index.shape[1]
    r = W_rel0.shape[0]
    nch = d // _CW

    tot = -(-e // (_NS * _BLK * 16)) * 16
    epb0 = max(8, int(round(0.7 * tot / 8)) * 8)
    epb1 = tot - epb0
    nrows = _NS * tot + (epb0 - epb1)
    pad = nrows * _BLK - e
    src = jnp.concatenate([edge_index[0], jnp.zeros((pad,), jnp.int32)])
    dst = jnp.concatenate([edge_index[1], jnp.full((pad,), n, jnp.int32)])
    typ = jnp.concatenate([edge_type, jnp.full((pad,), r - 1, jnp.int32)])
    src2 = src.reshape(nrows, _BLK)
    dst2 = dst.reshape(nrows, _BLK)
    typ2 = typ.reshape(nrows, _BLK)

    xc = [lax.slice_in_dim(x, c * _CW, (c + 1) * _CW, axis=1)
          for c in range(nch)]

    sc_agg0 = _make_sc_agg(n, r, d, epb0, epb1, with_counts=True)
    sums0, cnt = sc_agg0(src2, dst2, typ2, *xc)

    dense0 = _make_dense(n, r, d, emit_chunks=True)
    x1, *x1c = dense0(x, *([sums0] * r), *([cnt] * r), W_rel0, W_root0,
                      b0.reshape(1, d), g0.reshape(1, d), be0.reshape(1, d))

    sc_agg1 = _make_sc_agg(n, r, d, epb0, epb1, with_counts=False)
    sums1 = sc_agg1(src2, dst2, typ2, *x1c)
    if isinstance(sums1, (list, tuple)):
        sums1 = sums1[0]

    dense1 = _make_dense(n, r, d, emit_chunks=False)
    out = dense1(x1, *([sums1] * r), *([cnt] * r), W_rel1, W_root1,
                 b1.reshape(1, d), g1.reshape(1, d), be1.reshape(1, d))
    if isinstance(out, (list, tuple)):
        out = out[0]
    return out
